# baseline (speedup 1.0000x reference)
import functools

import jax
import jax.numpy as jnp
from jax import lax
from jax.experimental import pallas as pl
from jax.experimental.pallas import tpu as pltpu
from jax.experimental.pallas import tpu_sc as plsc

N = 1_000_000
B = 16_384
NC = 2
NS = 16
NW = NC * NS
CHUNK = 31_248
TAIL = N - NW * CHUNK
BPW = B // NW

_mesh = plsc.VectorSubcoreMesh(core_axis_name="c", subcore_axis_name="s")


@functools.partial(
    pl.kernel,
    mesh=_mesh,
    out_type=[
        jax.ShapeDtypeStruct((NW, 16), jnp.float32),
        jax.ShapeDtypeStruct((B,), jnp.float32),
    ],
    scratch_types=[
        pltpu.VMEM((CHUNK,), jnp.float32),
        pltpu.VMEM((TAIL,), jnp.float32),
        pltpu.VMEM((16,), jnp.float32),
        pltpu.VMEM((BPW,), jnp.int32),
        pltpu.VMEM((BPW,), jnp.float32),
        pltpu.SemaphoreType.DMA,
    ],
)
def _sc_sum_gather(probs_hbm, idx_hbm, partials_hbm, gathered_hbm,
                   buf, tailbuf, acc_v, idx_v, vals_v, sem):
    wid = lax.axis_index("s") * NC + lax.axis_index("c")

    gbase = wid * BPW
    pltpu.sync_copy(idx_hbm.at[pl.ds(gbase, BPW)], idx_v)
    gather = pltpu.async_copy(probs_hbm.at[idx_v], vals_v, sem)

    base = wid * CHUNK
    pltpu.sync_copy(probs_hbm.at[pl.ds(base, CHUNK)], buf)

    def body(i, acc):
        return acc + buf[pl.ds(i * 16, 16)]

    acc = lax.fori_loop(0, CHUNK // 16, body, jnp.zeros((16,), jnp.float32))
    acc_v[...] = acc

    @pl.when(wid == NW - 1)
    def _():
        pltpu.sync_copy(probs_hbm.at[pl.ds(NW * CHUNK, TAIL)], tailbuf)
        t = (tailbuf[pl.ds(0, 16)] + tailbuf[pl.ds(16, 16)]
             + tailbuf[pl.ds(32, 16)] + tailbuf[pl.ds(48, 16)])
        acc_v[...] = acc_v[...] + t

    pltpu.sync_copy(acc_v, partials_hbm.at[wid])

    gather.wait()
    pltpu.sync_copy(vals_v, gathered_hbm.at[pl.ds(gbase, BPW)])


def _combine_body(partials_ref, g_ref, out_ref):
    total = jnp.sum(partials_ref[...])
    out_ref[...] = jnp.log(g_ref[...]) - jnp.log(total)


_combine = pl.pallas_call(
    _combine_body,
    out_shape=jax.ShapeDtypeStruct((128, 128), jnp.float32),
)


@jax.jit
def kernel(probs, indices):
    partials, gathered = _sc_sum_gather(probs, indices.astype(jnp.int32))
    out2 = _combine(partials.reshape(4, 128), gathered.reshape(128, 128))
    return out2.reshape(B)


# speedup vs baseline: 1.2443x; 1.2443x over previous
import functools

import jax
import jax.numpy as jnp
from jax import lax
from jax.experimental import pallas as pl
from jax.experimental.pallas import tpu as pltpu
from jax.experimental.pallas import tpu_sc as plsc

N = 1_000_000
B = 16_384
NC = 2
NS = 16
NW = NC * NS
TSUM = 62_464
NCH = 4
CH = TSUM // NCH
TAIL = N - NS * TSUM
BPW = B // NW

_LN2 = 0.6931471805599453
_SQRT2 = 1.4142135623730951

_mesh = plsc.VectorSubcoreMesh(core_axis_name="c", subcore_axis_name="s")


def _vlog(x):
    bits = lax.bitcast_convert_type(x, jnp.int32)
    e = lax.shift_right_logical(bits, 23) - 127
    m = lax.bitcast_convert_type((bits & 0x7FFFFF) | 0x3F800000, jnp.float32)
    big = m >= _SQRT2
    m = jnp.where(big, m * 0.5, m)
    ef = (e + jnp.where(big, 1, 0)).astype(jnp.float32)
    s = (m - 1.0) / (m + 1.0)
    z = s * s
    p = 2.0 * s * (1.0 + z * (1.0 / 3.0 + z * (1.0 / 5.0 + z * (1.0 / 7.0))))
    return ef * _LN2 + p


@functools.partial(
    pl.kernel,
    mesh=_mesh,
    out_type=jax.ShapeDtypeStruct((B,), jnp.float32),
    scratch_types=[
        pltpu.VMEM((CH,), jnp.float32),
        pltpu.VMEM((CH,), jnp.float32),
        pltpu.VMEM((TAIL,), jnp.float32),
        pltpu.VMEM((16,), jnp.float32),
        pltpu.VMEM((NS, 16), jnp.float32),
        pltpu.VMEM((BPW,), jnp.int32),
        pltpu.VMEM((BPW,), jnp.float32),
        pltpu.VMEM((BPW,), jnp.float32),
        pltpu.VMEM_SHARED((NS, 16), jnp.float32),
        pltpu.SemaphoreType.DMA,
        pltpu.SemaphoreType.DMA,
        pltpu.SemaphoreType.DMA,
    ],
)
def _sc_logprob(probs_hbm, idx_hbm, out_hbm,
                buf0, buf1, tailbuf, acc_v, part_v, idx_v, vals_v,
                out_v, shared, sem0, sem1, gsem):
    cid = lax.axis_index("c")
    sid = lax.axis_index("s")
    wid = sid * NC + cid

    gbase = wid * BPW
    pltpu.sync_copy(idx_hbm.at[pl.ds(gbase, BPW)], idx_v)
    gather = pltpu.async_copy(probs_hbm.at[idx_v], vals_v, gsem)

    base = sid * TSUM
    bufs, sems = [buf0, buf1], [sem0, sem1]
    copies = [None, None]
    copies[0] = pltpu.async_copy(probs_hbm.at[pl.ds(base, CH)], buf0, sem0)
    accs = tuple(jnp.zeros((16,), jnp.float32) for _ in range(8))
    for k in range(NCH):
        if k + 1 < NCH:
            copies[(k + 1) % 2] = pltpu.async_copy(
                probs_hbm.at[pl.ds(base + (k + 1) * CH, CH)],
                bufs[(k + 1) % 2], sems[(k + 1) % 2])
        copies[k % 2].wait()
        buf = bufs[k % 2]

        def body(i, a, buf=buf):
            o = i * 128
            return tuple(a[j] + buf[pl.ds(o + j * 16, 16)] for j in range(8))

        accs = lax.fori_loop(0, CH // 128, body, accs)

    acc = ((accs[0] + accs[1]) + (accs[2] + accs[3])) + \
          ((accs[4] + accs[5]) + (accs[6] + accs[7]))
    acc_v[...] = acc

    @pl.when(sid == NS - 1)
    def _():
        pltpu.sync_copy(probs_hbm.at[pl.ds(NS * TSUM, TAIL)], tailbuf)
        t = jnp.zeros((16,), jnp.float32)
        for j in range(TAIL // 16):
            t = t + tailbuf[pl.ds(j * 16, 16)]
        acc_v[...] = acc_v[...] + t

    pltpu.sync_copy(acc_v, shared.at[sid])
    plsc.subcore_barrier()
    pltpu.sync_copy(shared, part_v)
    tot16 = part_v[0]
    for r in range(1, NS):
        tot16 = tot16 + part_v[r]
    s = tot16[0]
    for j in range(1, 16):
        s = s + tot16[j]
    log_tot = _vlog(jnp.zeros((16,), jnp.float32) + s)

    gather.wait()

    def gbody(i, carry):
        x = vals_v[pl.ds(i * 16, 16)]
        out_v[pl.ds(i * 16, 16)] = _vlog(x) - log_tot
        return carry

    lax.fori_loop(0, BPW // 16, gbody, 0)
    pltpu.sync_copy(out_v, out_hbm.at[pl.ds(gbase, BPW)])


@jax.jit
def kernel(probs, indices):
    return _sc_logprob(probs, indices.astype(jnp.int32))


# speedup vs baseline: 1.2509x; 1.0052x over previous
import functools

import jax
import jax.numpy as jnp
from jax import lax
from jax.experimental import pallas as pl
from jax.experimental.pallas import tpu as pltpu
from jax.experimental.pallas import tpu_sc as plsc

N = 1_000_000
B = 16_384
NC = 2
NS = 16
NW = NC * NS
TSUM = 62_464
NCH = 4
CH = TSUM // NCH
TAIL = N - NS * TSUM
BPW = B // NW

_LN2 = 0.6931471805599453
_SQRT2 = 1.4142135623730951

_mesh = plsc.VectorSubcoreMesh(core_axis_name="c", subcore_axis_name="s")


def _vlog(x):
    bits = lax.bitcast_convert_type(x, jnp.int32)
    e = lax.shift_right_logical(bits, 23) - 127
    m = lax.bitcast_convert_type((bits & 0x7FFFFF) | 0x3F800000, jnp.float32)
    big = m >= _SQRT2
    m = jnp.where(big, m * 0.5, m)
    ef = (e + jnp.where(big, 1, 0)).astype(jnp.float32)
    s = (m - 1.0) / (m + 1.0)
    z = s * s
    p = 2.0 * s * (1.0 + z * (1.0 / 3.0 + z * (1.0 / 5.0 + z * (1.0 / 7.0))))
    return ef * _LN2 + p


@functools.partial(
    pl.kernel,
    mesh=_mesh,
    out_type=jax.ShapeDtypeStruct((B,), jnp.float32),
    scratch_types=[
        pltpu.VMEM((CH,), jnp.float32),
        pltpu.VMEM((CH,), jnp.float32),
        pltpu.VMEM((TAIL,), jnp.float32),
        pltpu.VMEM((16,), jnp.float32),
        pltpu.VMEM((NS * 16,), jnp.float32),
        pltpu.VMEM((BPW,), jnp.int32),
        pltpu.VMEM((BPW,), jnp.float32),
        pltpu.VMEM((BPW,), jnp.float32),
        pltpu.VMEM_SHARED((NS * 16,), jnp.float32),
        pltpu.SemaphoreType.DMA,
        pltpu.SemaphoreType.DMA,
        pltpu.SemaphoreType.DMA,
    ],
)
def _sc_logprob(probs_hbm, idx_hbm, out_hbm,
                buf0, buf1, tailbuf, acc_v, part_v, idx_v, vals_v,
                out_v, shared, sem0, sem1, gsem):
    cid = lax.axis_index("c")
    sid = lax.axis_index("s")
    wid = sid * NC + cid

    gbase = wid * BPW
    pltpu.sync_copy(idx_hbm.at[pl.ds(gbase, BPW)], idx_v)
    gather = pltpu.async_copy(probs_hbm.at[idx_v], vals_v, gsem)

    base = sid * TSUM
    bufs, sems = [buf0, buf1], [sem0, sem1]
    copies = [None, None]
    copies[0] = pltpu.async_copy(probs_hbm.at[pl.ds(base, CH)], buf0, sem0)
    accs = tuple(jnp.zeros((16,), jnp.float32) for _ in range(8))
    for k in range(NCH):
        if k + 1 < NCH:
            copies[(k + 1) % 2] = pltpu.async_copy(
                probs_hbm.at[pl.ds(base + (k + 1) * CH, CH)],
                bufs[(k + 1) % 2], sems[(k + 1) % 2])
        copies[k % 2].wait()
        buf = bufs[k % 2]

        def body(i, a, buf=buf):
            o = i * 128
            return tuple(a[j] + buf[pl.ds(o + j * 16, 16)] for j in range(8))

        accs = lax.fori_loop(0, CH // 128, body, accs)

    acc = ((accs[0] + accs[1]) + (accs[2] + accs[3])) + \
          ((accs[4] + accs[5]) + (accs[6] + accs[7]))
    acc_v[...] = acc

    @pl.when(sid == NS - 1)
    def _():
        pltpu.sync_copy(probs_hbm.at[pl.ds(NS * TSUM, TAIL)], tailbuf)
        t = jnp.zeros((16,), jnp.float32)
        for j in range(TAIL // 16):
            t = t + tailbuf[pl.ds(j * 16, 16)]
        acc_v[...] = acc_v[...] + t

    gather.wait()

    pltpu.sync_copy(acc_v, shared.at[pl.ds(sid * 16, 16)])
    plsc.subcore_barrier()
    pltpu.sync_copy(shared, part_v)
    tot16 = part_v[pl.ds(0, 16)]
    for r in range(1, NS):
        tot16 = tot16 + part_v[pl.ds(r * 16, 16)]
    s = tot16[0]
    for j in range(1, 16):
        s = s + tot16[j]
    log_tot = _vlog(jnp.zeros((16,), jnp.float32) + s)

    def gbody(i, carry):
        x = vals_v[pl.ds(i * 16, 16)]
        out_v[pl.ds(i * 16, 16)] = _vlog(x) - log_tot
        return carry

    lax.fori_loop(0, BPW // 16, gbody, 0)
    pltpu.sync_copy(out_v, out_hbm.at[pl.ds(gbase, BPW)])


@jax.jit
def kernel(probs, indices):
    return _sc_logprob(probs, indices.astype(jnp.int32))


# speedup vs baseline: 1.2649x; 1.0112x over previous
import functools

import jax
import jax.numpy as jnp
from jax import lax
from jax.experimental import pallas as pl
from jax.experimental.pallas import tpu as pltpu
from jax.experimental.pallas import tpu_sc as plsc

N = 1_000_000
B = 16_384
NC = 2
NS = 16
NW = NC * NS
TSUM = 62_464
NCH = 2
CH = TSUM // NCH
TAIL = N - NS * TSUM
BPW = B // NW

_LN2 = 0.6931471805599453
_SQRT2 = 1.4142135623730951

_mesh = plsc.VectorSubcoreMesh(core_axis_name="c", subcore_axis_name="s")


def _vlog(x):
    bits = lax.bitcast_convert_type(x, jnp.int32)
    e = lax.shift_right_logical(bits, 23) - 127
    m = lax.bitcast_convert_type((bits & 0x7FFFFF) | 0x3F800000, jnp.float32)
    big = m >= _SQRT2
    m = jnp.where(big, m * 0.5, m)
    ef = (e + jnp.where(big, 1, 0)).astype(jnp.float32)
    s = (m - 1.0) / (m + 1.0)
    z = s * s
    p = 2.0 * s * (1.0 + z * (1.0 / 3.0 + z * (1.0 / 5.0 + z * (1.0 / 7.0))))
    return ef * _LN2 + p


@functools.partial(
    pl.kernel,
    mesh=_mesh,
    out_type=jax.ShapeDtypeStruct((B,), jnp.float32),
    scratch_types=[
        pltpu.VMEM((CH,), jnp.float32),
        pltpu.VMEM((CH,), jnp.float32),
        pltpu.VMEM((TAIL,), jnp.float32),
        pltpu.VMEM((16,), jnp.float32),
        pltpu.VMEM((NS * 16,), jnp.float32),
        pltpu.VMEM((BPW,), jnp.int32),
        pltpu.VMEM((BPW,), jnp.float32),
        pltpu.VMEM((BPW,), jnp.float32),
        pltpu.VMEM_SHARED((NS * 16,), jnp.float32),
        pltpu.SemaphoreType.DMA,
        pltpu.SemaphoreType.DMA,
        pltpu.SemaphoreType.DMA,
    ],
)
def _sc_logprob(probs_hbm, idx_hbm, out_hbm,
                buf0, buf1, tailbuf, acc_v, part_v, idx_v, vals_v,
                out_v, shared, sem0, sem1, gsem):
    cid = lax.axis_index("c")
    sid = lax.axis_index("s")
    wid = sid * NC + cid

    base = sid * TSUM
    bufs, sems = [buf0, buf1], [sem0, sem1]
    copies = [None, None]
    copies[0] = pltpu.async_copy(probs_hbm.at[pl.ds(base, CH)], buf0, sem0)

    gbase = wid * BPW
    pltpu.sync_copy(idx_hbm.at[pl.ds(gbase, BPW)], idx_v)
    gather = pltpu.async_copy(probs_hbm.at[idx_v], vals_v, gsem)
    accs = tuple(jnp.zeros((16,), jnp.float32) for _ in range(8))
    for k in range(NCH):
        if k + 1 < NCH:
            copies[(k + 1) % 2] = pltpu.async_copy(
                probs_hbm.at[pl.ds(base + (k + 1) * CH, CH)],
                bufs[(k + 1) % 2], sems[(k + 1) % 2])
        copies[k % 2].wait()
        buf = bufs[k % 2]

        def body(i, a, buf=buf):
            o = i * 128
            return tuple(a[j] + buf[pl.ds(o + j * 16, 16)] for j in range(8))

        accs = lax.fori_loop(0, CH // 128, body, accs)

    acc = ((accs[0] + accs[1]) + (accs[2] + accs[3])) + \
          ((accs[4] + accs[5]) + (accs[6] + accs[7]))
    acc_v[...] = acc

    @pl.when(sid == NS - 1)
    def _():
        pltpu.sync_copy(probs_hbm.at[pl.ds(NS * TSUM, TAIL)], tailbuf)
        t = jnp.zeros((16,), jnp.float32)
        for j in range(TAIL // 16):
            t = t + tailbuf[pl.ds(j * 16, 16)]
        acc_v[...] = acc_v[...] + t

    gather.wait()

    pltpu.sync_copy(acc_v, shared.at[pl.ds(sid * 16, 16)])
    plsc.subcore_barrier()
    pltpu.sync_copy(shared, part_v)
    tot16 = part_v[pl.ds(0, 16)]
    for r in range(1, NS):
        tot16 = tot16 + part_v[pl.ds(r * 16, 16)]
    s = tot16[0]
    for j in range(1, 16):
        s = s + tot16[j]
    log_tot = _vlog(jnp.zeros((16,), jnp.float32) + s)

    def gbody(i, carry):
        x = vals_v[pl.ds(i * 16, 16)]
        out_v[pl.ds(i * 16, 16)] = _vlog(x) - log_tot
        return carry

    lax.fori_loop(0, BPW // 16, gbody, 0)
    pltpu.sync_copy(out_v, out_hbm.at[pl.ds(gbase, BPW)])


@jax.jit
def kernel(probs, indices):
    return _sc_logprob(probs, indices.astype(jnp.int32))
